# baseline (speedup 1.0000x reference)
import functools

import jax
import jax.numpy as jnp
from jax import lax
from jax.experimental import pallas as pl
from jax.experimental.pallas import tpu as pltpu, tpu_sc as plsc

VOCAB_SIZE = 100000
MAX_LEN = 200
EMBED_DIM = 64
BATCH = 4096

_NC = 2
_NS = 16
_NW = _NC * _NS
_PHASES = EMBED_DIM // _NW
_GROUPS = BATCH // 16


def _body(xt_hbm, tokt_hbm, post_hbm, out_hbm,
          row_v, idx_v, out_v, pos_v, sem_i, sem_o):
    wid = lax.axis_index("s") * _NC + lax.axis_index("c")

    def fire_idx(l, slot):
        pltpu.async_copy(xt_hbm.at[l], idx_v.at[slot], sem_i)

    def drain_idx(l, slot):
        pltpu.make_async_copy(xt_hbm.at[l], idx_v.at[slot], sem_i).wait()

    def fire_out(l, d, slot):
        pltpu.async_copy(out_v.at[slot], out_hbm.at[l, d], sem_o)

    def drain_out(l, d, slot):
        pltpu.make_async_copy(out_v.at[slot], out_hbm.at[l, d], sem_o).wait()

    def phase(p, carry):
        d = _PHASES * wid + p
        pltpu.sync_copy(tokt_hbm.at[d], row_v)
        pltpu.sync_copy(post_hbm.at[d], pos_v.at[pl.ds(0, MAX_LEN)])
        fire_idx(0, 0)

        def do_l(l, carry):
            slot = l % 2
            drain_idx(l, slot)

            @pl.when(l + 1 < MAX_LEN)
            def _():
                fire_idx(l + 1, 1 - slot)

            @pl.when(l >= 2)
            def _():
                drain_out(l - 2, d, slot)

            pvec = pos_v[pl.ds((l // 16) * 16, 16)]
            pos_b = lax.gather(
                pvec,
                jnp.full((16, 1), l % 16, jnp.int32),
                lax.GatherDimensionNumbers(offset_dims=(),
                                           collapsed_slice_dims=(0,),
                                           start_index_map=(0,)),
                (1,),
                mode=lax.GatherScatterMode.PROMISE_IN_BOUNDS)

            def gath(g, carry):
                iv = idx_v[slot, pl.ds(g * 16, 16)]
                vals = plsc.load_gather(row_v, [iv])
                out_v[slot, pl.ds(g * 16, 16)] = vals + pos_b
                return carry

            lax.fori_loop(0, _GROUPS, gath, 0, unroll=4)
            fire_out(l, d, slot)
            return carry

        lax.fori_loop(0, MAX_LEN, do_l, 0)
        drain_out(MAX_LEN - 2, d, 0)
        drain_out(MAX_LEN - 1, d, 1)
        return carry

    lax.fori_loop(0, _PHASES, phase, 0)


@jax.jit
def kernel(x, token_emb, pos_emb):
    xt = x.astype(jnp.int32).T
    tokt = token_emb.T
    post = pos_emb.T
    mesh = plsc.VectorSubcoreMesh(core_axis_name="c", subcore_axis_name="s")
    k = functools.partial(
        pl.kernel,
        out_type=jax.ShapeDtypeStruct((MAX_LEN, EMBED_DIM, BATCH),
                                      jnp.float32),
        mesh=mesh,
        scratch_types=[
            pltpu.VMEM((VOCAB_SIZE,), jnp.float32),
            pltpu.VMEM((2, BATCH), jnp.int32),
            pltpu.VMEM((2, BATCH), jnp.float32),
            pltpu.VMEM((256,), jnp.float32),
            pltpu.SemaphoreType.DMA,
            pltpu.SemaphoreType.DMA,
        ],
        compiler_params=pltpu.CompilerParams(use_tc_tiling_on_sc=False,
                                             needs_layout_passes=False),
    )(_body)
    out_t = k(xt, tokt, post)
    return out_t.transpose(2, 0, 1)


# speedup vs baseline: 2.7197x; 2.7197x over previous
import functools

import jax
import jax.numpy as jnp
from jax import lax
from jax.experimental import pallas as pl
from jax.experimental.pallas import tpu as pltpu, tpu_sc as plsc

VOCAB_SIZE = 100000
MAX_LEN = 200
EMBED_DIM = 64
BATCH = 4096

_NC = 2
_NS = 16
_NW = _NC * _NS
_GROUPS = BATCH // 16


def _splat(pos_v, dd, l):
    pvec = pos_v[dd, pl.ds((l // 16) * 16, 16)]
    return lax.gather(
        pvec,
        jnp.full((16, 1), l % 16, jnp.int32),
        lax.GatherDimensionNumbers(offset_dims=(),
                                   collapsed_slice_dims=(0,),
                                   start_index_map=(0,)),
        (1,),
        mode=lax.GatherScatterMode.PROMISE_IN_BOUNDS)


def _body(xt_hbm, tokt_hbm, post_hbm, out_hbm,
          row_v, idx_v, out_v, pos_v, sem_i, sem_o):
    wid = lax.axis_index("s") * _NC + lax.axis_index("c")
    d0 = 2 * wid

    pltpu.sync_copy(tokt_hbm.at[wid], row_v)
    pltpu.sync_copy(post_hbm.at[d0], pos_v.at[0, pl.ds(0, MAX_LEN)])
    pltpu.sync_copy(post_hbm.at[d0 + 1], pos_v.at[1, pl.ds(0, MAX_LEN)])

    def fire_idx(l, slot):
        pltpu.async_copy(xt_hbm.at[l], idx_v.at[slot], sem_i)

    def drain_idx(l, slot):
        pltpu.make_async_copy(xt_hbm.at[l], idx_v.at[slot], sem_i).wait()

    def out_dst(l, dd):
        d = d0 + dd
        return out_hbm.at[l, d // 8, :, d % 8, :]

    def fire_out(l, slot):
        for dd in range(2):
            pltpu.async_copy(out_v.at[slot, dd], out_dst(l, dd), sem_o)

    def drain_out(l, slot):
        for dd in range(2):
            pltpu.make_async_copy(out_v.at[slot, dd], out_dst(l, dd),
                                  sem_o).wait()

    fire_idx(0, 0)

    def do_l(l, carry):
        slot = l % 2
        drain_idx(l, slot)

        @pl.when(l + 1 < MAX_LEN)
        def _():
            fire_idx(l + 1, 1 - slot)

        @pl.when(l >= 2)
        def _():
            drain_out(l - 2, slot)

        pos_b0 = _splat(pos_v, 0, l)
        pos_b1 = _splat(pos_v, 1, l)

        @plsc.parallel_loop(0, _GROUPS, unroll=8)
        def gath(g):
            iv = idx_v[slot, pl.ds(g * 16, 16)]
            pk = plsc.load_gather(row_v, [iv])
            a, b = plsc.unpack(plsc.bitcast(pk, jnp.bfloat16),
                               format=plsc.PackFormat.INTERLEAVED,
                               preferred_element_type=jnp.float32)
            out_v[slot, 0, g // 8, pl.ds((g % 8) * 16, 16)] = a + pos_b0
            out_v[slot, 1, g // 8, pl.ds((g % 8) * 16, 16)] = b + pos_b1

        fire_out(l, slot)
        return carry

    lax.fori_loop(0, MAX_LEN, do_l, 0)
    drain_out(MAX_LEN - 2, 0)
    drain_out(MAX_LEN - 1, 1)


@jax.jit
def kernel(x, token_emb, pos_emb):
    xt = x.astype(jnp.int32).T
    tokt = jax.lax.bitcast_convert_type(
        token_emb.astype(jnp.bfloat16).reshape(VOCAB_SIZE, _NW, 2),
        jnp.int32).T
    post = pos_emb.T
    mesh = plsc.VectorSubcoreMesh(core_axis_name="c", subcore_axis_name="s")
    k = functools.partial(
        pl.kernel,
        out_type=jax.ShapeDtypeStruct(
            (MAX_LEN, EMBED_DIM // 8, BATCH // 128, 8, 128), jnp.float32),
        mesh=mesh,
        scratch_types=[
            pltpu.VMEM((VOCAB_SIZE,), jnp.int32),
            pltpu.VMEM((2, BATCH), jnp.int32),
            pltpu.VMEM((2, 2, BATCH // 128, 128), jnp.float32),
            pltpu.VMEM((2, 256), jnp.float32),
            pltpu.SemaphoreType.DMA,
            pltpu.SemaphoreType.DMA,
        ],
        compiler_params=pltpu.CompilerParams(use_tc_tiling_on_sc=False,
                                             needs_layout_passes=False),
    )(_body)
    out5 = k(xt, tokt, post)
    out_t = out5.transpose(0, 1, 3, 2, 4).reshape(MAX_LEN, EMBED_DIM, BATCH)
    return out_t.transpose(2, 0, 1)


# speedup vs baseline: 3.8581x; 1.4186x over previous
import functools

import jax
import jax.numpy as jnp
from jax import lax
from jax.experimental import pallas as pl
from jax.experimental.pallas import tpu as pltpu, tpu_sc as plsc

VOCAB_SIZE = 100000
MAX_LEN = 200
EMBED_DIM = 64
BATCH = 4096

_NC = 2
_NS = 16
_NW = _NC * _NS
_GROUPS = BATCH // 16
_CHUNK = 2000
_NCHUNK = VOCAB_SIZE // _CHUNK


def _splat(pos_v, dd, l):
    pvec = pos_v[dd, pl.ds((l // 16) * 16, 16)]
    return lax.gather(
        pvec,
        jnp.full((16, 1), l % 16, jnp.int32),
        lax.GatherDimensionNumbers(offset_dims=(),
                                   collapsed_slice_dims=(0,),
                                   start_index_map=(0,)),
        (1,),
        mode=lax.GatherScatterMode.PROMISE_IN_BOUNDS)


def _body(xt_hbm, tokt_hbm, post_hbm, out_hbm,
          row_v, idx_v, out_v, pos_v, sem_i, sem_o):
    wid = lax.axis_index("s") * _NC + lax.axis_index("c")
    d0 = 2 * wid

    pltpu.sync_copy(post_hbm.at[d0], pos_v.at[0, pl.ds(0, MAX_LEN)])
    pltpu.sync_copy(post_hbm.at[d0 + 1], pos_v.at[1, pl.ds(0, MAX_LEN)])

    def fire_chunk(c, q):
        for dd in range(2):
            pltpu.async_copy(
                tokt_hbm.at[d0 + dd, pl.ds(c * _CHUNK, _CHUNK)],
                idx_v.at[dd, pl.ds(q * (BATCH // 2), _CHUNK)], sem_i)

    def drain_chunk(c, q):
        for dd in range(2):
            pltpu.make_async_copy(
                tokt_hbm.at[d0 + dd, pl.ds(c * _CHUNK, _CHUNK)],
                idx_v.at[dd, pl.ds(q * (BATCH // 2), _CHUNK)], sem_i).wait()

    fire_chunk(0, 0)

    def pack_chunk(c, carry):
        q = c % 2
        drain_chunk(c, q)

        @pl.when(c + 1 < _NCHUNK)
        def _():
            fire_chunk(c + 1, 1 - q)

        @plsc.parallel_loop(0, _CHUNK // 16, unroll=8)
        def pk(g):
            base = q * (BATCH // 2) + g * 16
            a = plsc.bitcast(idx_v[0, pl.ds(base, 16)], jnp.float32)
            b = plsc.bitcast(idx_v[1, pl.ds(base, 16)], jnp.float32)
            packed = plsc.pack(a, b, format=plsc.PackFormat.INTERLEAVED)
            row_v[pl.ds(c * _CHUNK + g * 16, 16)] = plsc.bitcast(
                packed, jnp.int32)

        return carry

    lax.fori_loop(0, _NCHUNK, pack_chunk, 0)

    def fire_idx(l, slot):
        pltpu.async_copy(xt_hbm.at[l], idx_v.at[slot], sem_i)

    def drain_idx(l, slot):
        pltpu.make_async_copy(xt_hbm.at[l], idx_v.at[slot], sem_i).wait()

    def out_dst(l, dd):
        d = d0 + dd
        return out_hbm.at[l, d // 8, :, d % 8, :]

    def fire_out(l, slot):
        for dd in range(2):
            pltpu.async_copy(out_v.at[slot, dd], out_dst(l, dd), sem_o)

    def drain_out(l, slot):
        for dd in range(2):
            pltpu.make_async_copy(out_v.at[slot, dd], out_dst(l, dd),
                                  sem_o).wait()

    fire_idx(0, 0)

    def do_l(l, carry):
        slot = l % 2
        drain_idx(l, slot)

        @pl.when(l + 1 < MAX_LEN)
        def _():
            fire_idx(l + 1, 1 - slot)

        @pl.when(l >= 2)
        def _():
            drain_out(l - 2, slot)

        pos_b0 = _splat(pos_v, 0, l)
        pos_b1 = _splat(pos_v, 1, l)

        @plsc.parallel_loop(0, _GROUPS, unroll=8)
        def gath(g):
            iv = idx_v[slot, pl.ds(g * 16, 16)]
            pk = plsc.load_gather(row_v, [iv])
            a, b = plsc.unpack(plsc.bitcast(pk, jnp.bfloat16),
                               format=plsc.PackFormat.INTERLEAVED,
                               preferred_element_type=jnp.float32)
            out_v[slot, 0, g // 8, pl.ds((g % 8) * 16, 16)] = a + pos_b0
            out_v[slot, 1, g // 8, pl.ds((g % 8) * 16, 16)] = b + pos_b1

        fire_out(l, slot)
        return carry

    lax.fori_loop(0, MAX_LEN, do_l, 0)
    drain_out(MAX_LEN - 2, 0)
    drain_out(MAX_LEN - 1, 1)


@jax.jit
def kernel(x, token_emb, pos_emb):
    xt = x.astype(jnp.int32).T
    tokt = jax.lax.bitcast_convert_type(token_emb, jnp.int32).T
    post = pos_emb.T
    mesh = plsc.VectorSubcoreMesh(core_axis_name="c", subcore_axis_name="s")
    k = functools.partial(
        pl.kernel,
        out_type=jax.ShapeDtypeStruct(
            (MAX_LEN, EMBED_DIM // 8, BATCH // 128, 8, 128), jnp.float32),
        mesh=mesh,
        scratch_types=[
            pltpu.VMEM((VOCAB_SIZE,), jnp.int32),
            pltpu.VMEM((2, BATCH), jnp.int32),
            pltpu.VMEM((2, 2, BATCH // 128, 128), jnp.float32),
            pltpu.VMEM((2, 256), jnp.float32),
            pltpu.SemaphoreType.DMA,
            pltpu.SemaphoreType.DMA,
        ],
        compiler_params=pltpu.CompilerParams(use_tc_tiling_on_sc=False,
                                             needs_layout_passes=False),
    )(_body)
    out5 = k(xt, tokt, post)
    out_t = out5.transpose(0, 1, 3, 2, 4).reshape(MAX_LEN, EMBED_DIM, BATCH)
    return out_t.transpose(2, 0, 1)


# speedup vs baseline: 4.8354x; 1.2533x over previous
import functools

import jax
import jax.numpy as jnp
from jax import lax
from jax.experimental import pallas as pl
from jax.experimental.pallas import tpu as pltpu, tpu_sc as plsc

VOCAB_SIZE = 100000
MAX_LEN = 200
EMBED_DIM = 64
BATCH = 4096

_NC = 2
_NS = 16
_NW = _NC * _NS
_GROUPS = BATCH // 16
_CHUNK = 2000
_NCHUNK = VOCAB_SIZE // _CHUNK


def _splat(pos_v, dd, l):
    pvec = pos_v[dd, pl.ds((l // 16) * 16, 16)]
    return lax.gather(
        pvec,
        jnp.full((16, 1), l % 16, jnp.int32),
        lax.GatherDimensionNumbers(offset_dims=(),
                                   collapsed_slice_dims=(0,),
                                   start_index_map=(0,)),
        (1,),
        mode=lax.GatherScatterMode.PROMISE_IN_BOUNDS)


def _body(xt_hbm, tokt_hbm, post_hbm, out_hbm,
          row_v, idx_v, out_v, pos_v, sem_i, sem_o):
    wid = lax.axis_index("s") * _NC + lax.axis_index("c")
    d0 = 2 * wid

    pltpu.sync_copy(post_hbm.at[d0], pos_v.at[0, pl.ds(0, MAX_LEN)])
    pltpu.sync_copy(post_hbm.at[d0 + 1], pos_v.at[1, pl.ds(0, MAX_LEN)])

    def fire_chunk(c, q):
        for dd in range(2):
            pltpu.async_copy(
                tokt_hbm.at[d0 + dd, pl.ds(c * _CHUNK, _CHUNK)],
                idx_v.at[dd, pl.ds(q * (BATCH // 2), _CHUNK)], sem_i)

    def drain_chunk(c, q):
        for dd in range(2):
            pltpu.make_async_copy(
                tokt_hbm.at[d0 + dd, pl.ds(c * _CHUNK, _CHUNK)],
                idx_v.at[dd, pl.ds(q * (BATCH // 2), _CHUNK)], sem_i).wait()

    fire_chunk(0, 0)

    def pack_chunk(c, carry):
        q = c % 2
        drain_chunk(c, q)

        @pl.when(c + 1 < _NCHUNK)
        def _():
            fire_chunk(c + 1, 1 - q)

        @plsc.parallel_loop(0, _CHUNK // 16, unroll=8)
        def pk(g):
            base = q * (BATCH // 2) + g * 16
            a = plsc.bitcast(idx_v[0, pl.ds(base, 16)], jnp.float32)
            b = plsc.bitcast(idx_v[1, pl.ds(base, 16)], jnp.float32)
            packed = plsc.pack(a, b, format=plsc.PackFormat.INTERLEAVED)
            row_v[pl.ds(c * _CHUNK + g * 16, 16)] = plsc.bitcast(
                packed, jnp.int32)

        return carry

    lax.fori_loop(0, _NCHUNK, pack_chunk, 0)

    def fire_idx(l):
        pltpu.async_copy(xt_hbm.at[l], idx_v.at[l % 3], sem_i)

    def drain_idx(l):
        pltpu.make_async_copy(xt_hbm.at[l], idx_v.at[l % 3], sem_i).wait()

    def out_dst(l):
        return out_hbm.at[l, d0 // 8, :, pl.ds(d0 % 8, 2), :]

    def fire_out(l, slot):
        pltpu.async_copy(out_v.at[slot], out_dst(l), sem_o)

    def drain_out(l, slot):
        pltpu.make_async_copy(out_v.at[slot], out_dst(l), sem_o).wait()

    fire_idx(0)
    fire_idx(1)

    def do_l(l, carry):
        slot = l % 2
        drain_idx(l)

        @pl.when(l + 2 < MAX_LEN)
        def _():
            fire_idx(l + 2)

        @pl.when(l >= 2)
        def _():
            drain_out(l - 2, slot)

        pos_b0 = _splat(pos_v, 0, l)
        pos_b1 = _splat(pos_v, 1, l)
        islot = l % 3

        @plsc.parallel_loop(0, _GROUPS, unroll=8)
        def gath(g):
            iv = idx_v[islot, pl.ds(g * 16, 16)]
            pk = plsc.load_gather(row_v, [iv])
            a, b = plsc.unpack(plsc.bitcast(pk, jnp.bfloat16),
                               format=plsc.PackFormat.INTERLEAVED,
                               preferred_element_type=jnp.float32)
            out_v[slot, g // 8, 0, pl.ds((g % 8) * 16, 16)] = a + pos_b0
            out_v[slot, g // 8, 1, pl.ds((g % 8) * 16, 16)] = b + pos_b1

        fire_out(l, slot)
        return carry

    lax.fori_loop(0, MAX_LEN, do_l, 0)
    drain_out(MAX_LEN - 2, 0)
    drain_out(MAX_LEN - 1, 1)


@jax.jit
def kernel(x, token_emb, pos_emb):
    xt = x.astype(jnp.int32).T
    tokt = jax.lax.bitcast_convert_type(token_emb, jnp.int32).T
    post = pos_emb.T
    mesh = plsc.VectorSubcoreMesh(core_axis_name="c", subcore_axis_name="s")
    k = functools.partial(
        pl.kernel,
        out_type=jax.ShapeDtypeStruct(
            (MAX_LEN, EMBED_DIM // 8, BATCH // 128, 8, 128), jnp.float32),
        mesh=mesh,
        scratch_types=[
            pltpu.VMEM((VOCAB_SIZE,), jnp.int32),
            pltpu.VMEM((3, BATCH), jnp.int32),
            pltpu.VMEM((2, BATCH // 128, 2, 128), jnp.float32),
            pltpu.VMEM((2, 256), jnp.float32),
            pltpu.SemaphoreType.DMA,
            pltpu.SemaphoreType.DMA,
        ],
        compiler_params=pltpu.CompilerParams(use_tc_tiling_on_sc=False,
                                             needs_layout_passes=False),
    )(_body)
    out5 = k(xt, tokt, post)
    out_t = out5.transpose(0, 1, 3, 2, 4).reshape(MAX_LEN, EMBED_DIM, BATCH)
    return out_t.transpose(2, 0, 1)
